# baseline (speedup 1.0000x reference)
import functools

import jax
import jax.numpy as jnp
from jax import lax
from jax.experimental import pallas as pl
from jax.experimental.pallas import tpu as pltpu
from jax.experimental.pallas import tpu_sc as plsc

NC = 2
NS = 16
NW = NC * NS

BATCH = 16384
D_EMB = 123
D_OUT = 128
B_PER_W = BATCH // NW
N_CHUNK = B_PER_W // 128


NUM_ITEMS = 100000
PAD_BLK = 4000


def _pad_body(b1_ref, w2_ref, b2_ref, w3_ref, b3_ref, emb_ref, out_ref):
    t = b1_ref[...].reshape(1, 30) @ w2_ref[...] + b2_ref[...].reshape(1, 30)
    h = t @ w3_ref[...] + b3_ref[...].reshape(1, 5)
    out_ref[:, :D_EMB] = emb_ref[...]
    out_ref[:, D_EMB:] = jnp.broadcast_to(h, (PAD_BLK, 5))


@jax.jit
def _pad_call(b1, W2, b2, W3, b3, W_emb):
    return pl.pallas_call(
        _pad_body,
        grid=(NUM_ITEMS // PAD_BLK,),
        in_specs=[
            pl.BlockSpec(b1.shape, lambda i: (0,)),
            pl.BlockSpec(W2.shape, lambda i: (0, 0)),
            pl.BlockSpec(b2.shape, lambda i: (0,)),
            pl.BlockSpec(W3.shape, lambda i: (0, 0)),
            pl.BlockSpec(b3.shape, lambda i: (0,)),
            pl.BlockSpec((PAD_BLK, D_EMB), lambda i: (i, 0)),
        ],
        out_specs=pl.BlockSpec((PAD_BLK, D_OUT), lambda i: (i, 0)),
        out_shape=jax.ShapeDtypeStruct((NUM_ITEMS, D_OUT), jnp.float32),
    )(b1, W2, b2, W3, b3, W_emb)


def _sc_body(idx_hbm, table_hbm, out_hbm, idx_v, out_v,
             gsem_0, gsem_1, gsem_2, gsem_3, osem):
    wid = lax.axis_index("s") * NC + lax.axis_index("c")
    base = wid * B_PER_W
    gsems = (gsem_0, gsem_1, gsem_2, gsem_3)

    pltpu.sync_copy(idx_hbm.at[pl.ds(base, B_PER_W)], idx_v)
    gathers = []
    for j in range(N_CHUNK):
        sl = pl.ds(j * 128, 128)
        gathers.append(
            pltpu.async_copy(table_hbm.at[idx_v.at[sl]], out_v.at[sl],
                             gsems[j]))

    writes = []
    for j in range(N_CHUNK):
        gathers[j].wait()
        csl = pl.ds(j * 128, 128)
        writes.append(
            pltpu.async_copy(out_v.at[csl],
                             out_hbm.at[pl.ds(base + j * 128, 128)], osem))
    for w in writes:
        w.wait()


@jax.jit
def _sc_call(idx, table_pad):
    mesh = plsc.VectorSubcoreMesh(core_axis_name="c", subcore_axis_name="s")
    run = functools.partial(
        pl.kernel,
        out_type=jax.ShapeDtypeStruct((BATCH, D_OUT), jnp.float32),
        mesh=mesh,
        scratch_types=[
            pltpu.VMEM((B_PER_W,), jnp.int32),
            pltpu.VMEM((B_PER_W, D_OUT), jnp.float32),
            pltpu.SemaphoreType.DMA,
            pltpu.SemaphoreType.DMA,
            pltpu.SemaphoreType.DMA,
            pltpu.SemaphoreType.DMA,
            pltpu.SemaphoreType.DMA,
        ],
    )(_sc_body)
    return run(idx, table_pad)


def kernel(item_inputs, W_emb, W1, b1, W2, b2, W3, b3):
    del W1
    table_pad = _pad_call(b1, W2, b2, W3, b3, W_emb)
    return _sc_call(item_inputs, table_pad)


# speedup vs baseline: 1.0793x; 1.0793x over previous
import functools

import jax
import jax.numpy as jnp
from jax import lax
from jax.experimental import pallas as pl
from jax.experimental.pallas import tpu as pltpu
from jax.experimental.pallas import tpu_sc as plsc

NC = 2
NS = 16
NW = NC * NS

BATCH = 16384
D_EMB = 123
D_OUT = 128
B_PER_W = BATCH // NW
N_CHUNK = B_PER_W // 128


def _h_body(b1_ref, w2_ref, b2_ref, w3_ref, b3_ref, h_ref):
    t = b1_ref[:].reshape(1, 30) @ w2_ref[:] + b2_ref[:].reshape(1, 30)
    h_ref[:] = (t @ w3_ref[:] + b3_ref[:].reshape(1, 5)).reshape(5)


@jax.jit
def _h_call(b1, W2, b2, W3, b3):
    return pl.pallas_call(
        _h_body,
        out_shape=jax.ShapeDtypeStruct((5,), jnp.float32),
    )(b1, W2, b2, W3, b3)


def _sc_body(idx_hbm, table_hbm, out_hbm, idx_v, out_v,
             gsem_0, gsem_1, gsem_2, gsem_3, osem):
    wid = lax.axis_index("s") * NC + lax.axis_index("c")
    base = wid * B_PER_W
    gsems = (gsem_0, gsem_1, gsem_2, gsem_3)

    pltpu.sync_copy(idx_hbm.at[pl.ds(base, B_PER_W)], idx_v)
    gathers = []
    for j in range(N_CHUNK):
        sl = pl.ds(j * 128, 128)
        gathers.append(
            pltpu.async_copy(table_hbm.at[idx_v.at[sl]], out_v.at[sl],
                             gsems[j]))

    writes = []
    for j in range(N_CHUNK):
        gathers[j].wait()
        csl = pl.ds(j * 128, 128)
        writes.append(
            pltpu.async_copy(out_v.at[csl],
                             out_hbm.at[pl.ds(base + j * 128, 128)], osem))
    for w in writes:
        w.wait()


@jax.jit
def _sc_call(idx, table_pad):
    mesh = plsc.VectorSubcoreMesh(core_axis_name="c", subcore_axis_name="s")
    run = functools.partial(
        pl.kernel,
        out_type=jax.ShapeDtypeStruct((BATCH, D_OUT), jnp.float32),
        mesh=mesh,
        scratch_types=[
            pltpu.VMEM((B_PER_W,), jnp.int32),
            pltpu.VMEM((B_PER_W, D_OUT), jnp.float32),
            pltpu.SemaphoreType.DMA,
            pltpu.SemaphoreType.DMA,
            pltpu.SemaphoreType.DMA,
            pltpu.SemaphoreType.DMA,
            pltpu.SemaphoreType.DMA,
        ],
    )(_sc_body)
    return run(idx, table_pad)


def kernel(item_inputs, W_emb, W1, b1, W2, b2, W3, b3):
    del W1
    h = _h_call(b1, W2, b2, W3, b3)
    table_pad = jnp.concatenate(
        [W_emb, jnp.broadcast_to(h, (W_emb.shape[0], 5))], axis=1)
    return _sc_call(item_inputs, table_pad)
